# Initial kernel scaffold; baseline (speedup 1.0000x reference)
import jax
import jax.numpy as jnp
from jax import lax
from jax.experimental import pallas as pl
from jax.experimental.pallas import tpu as pltpu
from jax.experimental.pallas import tpu_sc as plsc

N = 10000
E = 320000
D = 128
LANES = 16
EB = 128
ROWS = E // EB
NC = 2
NS = 16
NW = NC * NS
RPW = ROWS // NW
EXTRA = ROWS - RPW * NW
RPS = N // NS


def _sc_body(x_hbm, src_hbm, dst_hbm, w_hbm, agg_out, cnt_out,
             src_b, dst_b, w_b, rows, ones16, zeros16, agg_sh, cnt_sh, sem):
    cid = lax.axis_index("c")
    sid = lax.axis_index("s")
    wid = sid * NC + cid

    def init_row(e, carry):
        for j in range(D // LANES):
            rows[e, pl.ds(j * LANES, LANES)] = jnp.zeros((LANES,), jnp.float32)
        ones16[e, :] = jnp.ones((LANES,), jnp.float32)
        zeros16[e, :] = jnp.zeros((LANES,), jnp.float32)
        return carry
    lax.fori_loop(0, EB, init_row, 0)

    for k in range(RPS // 125):
        off = sid * RPS + k * 125
        pltpu.sync_copy(rows.at[pl.ds(0, 125)], agg_sh.at[pl.ds(off, 125)])
        pltpu.sync_copy(zeros16.at[pl.ds(0, 125)], cnt_sh.at[pl.ds(off, 125)])

    base = wid * RPW
    pltpu.sync_copy(src_hbm.at[pl.ds(base, RPW)], src_b.at[pl.ds(0, RPW)])
    pltpu.sync_copy(dst_hbm.at[pl.ds(base, RPW)], dst_b.at[pl.ds(0, RPW)])
    pltpu.sync_copy(w_hbm.at[pl.ds(base, RPW)], w_b.at[pl.ds(0, RPW)])

    @pl.when(wid < EXTRA)
    def _():
        xr = NW * RPW + wid
        pltpu.sync_copy(src_hbm.at[xr], src_b.at[RPW])
        pltpu.sync_copy(dst_hbm.at[xr], dst_b.at[RPW])
        pltpu.sync_copy(w_hbm.at[xr], w_b.at[RPW])

    nb = RPW + jnp.where(wid < EXTRA, 1, 0)

    plsc.subcore_barrier()

    def batch_body(b, carry):
        pltpu.async_copy(x_hbm.at[src_b.at[b]], rows, sem).wait()

        def scale_row(e, c2):
            ws = w_b[b, e]
            for j in range(D // LANES):
                sl = pl.ds(j * LANES, LANES)
                rows[e, sl] = rows[e, sl] * ws
            return c2
        lax.fori_loop(0, EB, scale_row, 0)

        pltpu.sync_copy(rows, agg_sh.at[dst_b.at[b]], add=True)
        pltpu.sync_copy(ones16, cnt_sh.at[dst_b.at[b]], add=True)
        return carry
    lax.fori_loop(0, nb, batch_body, 0)

    plsc.subcore_barrier()

    off = sid * RPS
    pltpu.sync_copy(agg_sh.at[pl.ds(off, RPS)], agg_out.at[cid, pl.ds(off, RPS)])
    pltpu.sync_copy(cnt_sh.at[pl.ds(off, RPS)], cnt_out.at[cid, pl.ds(off, RPS)])


def _sc_partials(x2d, src2d, dst2d, w2d):
    mesh = plsc.VectorSubcoreMesh(core_axis_name="c", subcore_axis_name="s")
    fn = pl.kernel(
        _sc_body,
        out_type=(
            jax.ShapeDtypeStruct((NC, N, D), jnp.float32),
            jax.ShapeDtypeStruct((NC, N, LANES), jnp.float32),
        ),
        mesh=mesh,
        scratch_types=[
            pltpu.VMEM((RPW + 1, EB), jnp.int32),
            pltpu.VMEM((RPW + 1, EB), jnp.int32),
            pltpu.VMEM((RPW + 1, EB), jnp.float32),
            pltpu.VMEM((EB, D), jnp.float32),
            pltpu.VMEM((EB, LANES), jnp.float32),
            pltpu.VMEM((EB, LANES), jnp.float32),
            pltpu.VMEM_SHARED((N, D), jnp.float32),
            pltpu.VMEM_SHARED((N, LANES), jnp.float32),
            pltpu.SemaphoreType.DMA,
        ],
    )
    return fn(x2d, src2d, dst2d, w2d)


def _tc_body(agg_ref, cnt_ref, x_ref, wr_ref, wroot_ref, br_ref, out_ref):
    agg = agg_ref[0] + agg_ref[1]
    cnt = cnt_ref[0, :, :1] + cnt_ref[1, :, :1]
    mean = agg / jnp.maximum(cnt, 1.0)
    out_ref[...] = (
        lax.dot_general(mean, wr_ref[...], (((1,), (1,)), ((), ())),
                        preferred_element_type=jnp.float32)
        + lax.dot_general(x_ref[...], wroot_ref[...], (((1,), (1,)), ((), ())),
                          preferred_element_type=jnp.float32)
        + br_ref[...]
    )


def _tc_combine(agg2, cnt2, x2d, W_rel, b_rel2, W_root):
    R = 1250
    return pl.pallas_call(
        _tc_body,
        grid=(N // R,),
        in_specs=[
            pl.BlockSpec((NC, R, D), lambda i: (0, i, 0)),
            pl.BlockSpec((NC, R, LANES), lambda i: (0, i, 0)),
            pl.BlockSpec((R, D), lambda i: (i, 0)),
            pl.BlockSpec((D, D), lambda i: (0, 0)),
            pl.BlockSpec((D, D), lambda i: (0, 0)),
            pl.BlockSpec((1, D), lambda i: (0, 0)),
        ],
        out_specs=pl.BlockSpec((R, D), lambda i: (i, 0)),
        out_shape=jax.ShapeDtypeStruct((N, D), jnp.float32),
    )(agg2, cnt2, x2d, W_rel, W_root, b_rel2)


@jax.jit
def kernel(x, index, weight, W_rel, b_rel, W_root):
    x2d = x.reshape(N, D)
    src2d = index[0].reshape(ROWS, EB)
    dst2d = index[1].reshape(ROWS, EB)
    w2d = weight.reshape(ROWS, EB)

    agg2, cnt2 = _sc_partials(x2d, src2d, dst2d, w2d)
    out = _tc_combine(agg2, cnt2, x2d, W_rel, b_rel.reshape(1, D), W_root)
    return out.reshape(1, N, D)


# speedup vs baseline: 1.8907x; 1.8907x over previous
import jax
import jax.numpy as jnp
from jax import lax
from jax.experimental import pallas as pl
from jax.experimental.pallas import tpu as pltpu
from jax.experimental.pallas import tpu_sc as plsc

N = 10000
E = 320000
D = 128
LANES = 16
EB = 64
NC = 2
NS = 16
NW = NC * NS
RPW = 160
CH = 16
NPAD = 10240
RPS = NPAD // NS


def _sc_body(scale, x_hbm, src_hbm, dst_hbm, w_hbm, agg_out,
             src_b, dst_b, w_b, rows, agg_sh, sem):
    cid = lax.axis_index("c")
    sid = lax.axis_index("s")
    wid = sid * NC + cid

    def init_row(e, carry):
        for j in range(D // LANES):
            rows[e, pl.ds(j * LANES, LANES)] = jnp.zeros((LANES,), jnp.float32)
        return carry
    lax.fori_loop(0, EB, init_row, 0)

    def build_idx(base_val):
        for j in range(EB // LANES):
            src_b[0, pl.ds(j * LANES, LANES)] = (
                lax.iota(jnp.int32, LANES) + base_val + j * LANES)

    off0 = sid * RPS
    for c in range(RPS // EB):
        build_idx(off0 + c * EB)
        pltpu.sync_copy(rows, agg_sh.at[src_b.at[0]])

    base = wid * RPW
    plsc.subcore_barrier()

    def chunk_body(c, carry):
        cbase = base + c * CH
        pltpu.sync_copy(src_hbm.at[pl.ds(cbase, CH)], src_b)
        pltpu.sync_copy(dst_hbm.at[pl.ds(cbase, CH)], dst_b)
        if scale:
            pltpu.sync_copy(w_hbm.at[pl.ds(cbase, CH)], w_b)

        def batch_body(b, c1):
            pltpu.async_copy(x_hbm.at[src_b.at[b]], rows, sem).wait()

            if scale:
                def scale_group(g, c2):
                    wv = w_b[b, pl.ds(g * LANES, LANES)]
                    e0 = g * LANES
                    for l in range(LANES):
                        ws = wv[l]
                        for j in range(D // LANES):
                            sl = pl.ds(j * LANES, LANES)
                            rows[e0 + l, sl] = rows[e0 + l, sl] * ws
                    return c2
                lax.fori_loop(0, EB // LANES, scale_group, 0)

            pltpu.sync_copy(rows, agg_sh.at[dst_b.at[b]], add=True)
            return c1
        lax.fori_loop(0, CH, batch_body, 0)
        return carry
    lax.fori_loop(0, RPW // CH, chunk_body, 0)

    plsc.subcore_barrier()

    for c in range(RPS // EB):
        coff = off0 + c * EB
        build_idx(coff)
        pltpu.sync_copy(agg_sh.at[src_b.at[0]], rows)
        pltpu.sync_copy(rows, agg_out.at[cid, pl.ds(coff, EB)])


def _make_sc(scale):
    import functools
    mesh = plsc.VectorSubcoreMesh(core_axis_name="c", subcore_axis_name="s")
    return pl.kernel(
        functools.partial(_sc_body, scale),
        out_type=jax.ShapeDtypeStruct((NC, NPAD, D), jnp.float32),
        mesh=mesh,
        scratch_types=[
            pltpu.VMEM((CH, EB), jnp.int32),
            pltpu.VMEM((CH, EB), jnp.int32),
            pltpu.VMEM((CH, EB), jnp.float32),
            pltpu.VMEM((EB, D), jnp.float32),
            pltpu.VMEM_SHARED((NPAD, D), jnp.float32),
            pltpu.SemaphoreType.DMA,
        ],
    )


def _tc_body(agg_ref, cnt_ref, x_ref, wr_ref, wroot_ref, br_ref, out_ref):
    agg = agg_ref[0] + agg_ref[1]
    cnt = cnt_ref[0, :, :1] + cnt_ref[1, :, :1]
    mean = agg / jnp.maximum(cnt, 1.0)
    out_ref[...] = (
        lax.dot_general(mean, wr_ref[...], (((1,), (1,)), ((), ())),
                        preferred_element_type=jnp.float32)
        + lax.dot_general(x_ref[...], wroot_ref[...], (((1,), (1,)), ((), ())),
                          preferred_element_type=jnp.float32)
        + br_ref[...]
    )


def _tc_combine(agg2, cnt2, x2d, W_rel, b_rel2, W_root):
    R = 1000
    return pl.pallas_call(
        _tc_body,
        grid=(N // R,),
        in_specs=[
            pl.BlockSpec((NC, R, D), lambda i: (0, i, 0)),
            pl.BlockSpec((NC, R, D), lambda i: (0, i, 0)),
            pl.BlockSpec((R, D), lambda i: (i, 0)),
            pl.BlockSpec((D, D), lambda i: (0, 0)),
            pl.BlockSpec((D, D), lambda i: (0, 0)),
            pl.BlockSpec((1, D), lambda i: (0, 0)),
        ],
        out_specs=pl.BlockSpec((R, D), lambda i: (i, 0)),
        out_shape=jax.ShapeDtypeStruct((N, D), jnp.float32),
    )(agg2, cnt2, x2d, W_rel, W_root, b_rel2)


@jax.jit
def kernel(x, index, weight, W_rel, b_rel, W_root):
    x2d = x.reshape(N, D)
    pad = NW * RPW * EB - E
    src2d = jnp.pad(index[0], (0, pad)).reshape(NW * RPW, EB)
    dst2d = jnp.pad(index[1], (0, pad), constant_values=N).reshape(NW * RPW, EB)
    w2d = jnp.pad(weight, (0, pad)).reshape(NW * RPW, EB)
    ones2d = jnp.ones((N, D), jnp.float32)

    agg2 = _make_sc(True)(x2d, src2d, dst2d, w2d)
    cnt2 = _make_sc(False)(ones2d, src2d, dst2d, w2d)
    out = _tc_combine(agg2, cnt2, x2d, W_rel, b_rel.reshape(1, D), W_root)
    return out.reshape(1, N, D)


# speedup vs baseline: 3.3228x; 1.7574x over previous
import jax
import jax.numpy as jnp
from jax import lax
from jax.experimental import pallas as pl
from jax.experimental.pallas import tpu as pltpu
from jax.experimental.pallas import tpu_sc as plsc

N = 10000
E = 320000
D = 128
LANES = 16
EB = 64
NC = 2
NS = 16
NW = NC * NS
RPW = 160
CH = 16
NPAD = 10240
RPS = NPAD // NS


def _sc_body(scale, x_hbm, src_hbm, dst_hbm, w_hbm, agg_out,
             src_b, dst_b, w_b, rows, agg_sh, sem):
    cid = lax.axis_index("c")
    sid = lax.axis_index("s")
    wid = sid * NC + cid

    def init_row(e, carry):
        for j in range(D // LANES):
            rows[e, pl.ds(j * LANES, LANES)] = jnp.zeros((LANES,), jnp.float32)
        return carry
    lax.fori_loop(0, EB, init_row, 0)

    def build_idx(base_val):
        for j in range(EB // LANES):
            src_b[0, pl.ds(j * LANES, LANES)] = (
                lax.iota(jnp.int32, LANES) + base_val + j * LANES)

    off0 = sid * RPS
    for c in range(RPS // EB):
        build_idx(off0 + c * EB)
        pltpu.sync_copy(rows, agg_sh.at[src_b.at[0]])

    if not scale:
        def ones_row(e, carry):
            for j in range(D // LANES):
                rows[e, pl.ds(j * LANES, LANES)] = jnp.ones((LANES,), jnp.float32)
            return carry
        lax.fori_loop(0, EB, ones_row, 0)

    base = wid * RPW
    plsc.subcore_barrier()

    def chunk_body(c, carry):
        cbase = base + c * CH
        pltpu.sync_copy(dst_hbm.at[pl.ds(cbase, CH)], dst_b)
        if scale:
            pltpu.sync_copy(src_hbm.at[pl.ds(cbase, CH)], src_b)
            pltpu.sync_copy(w_hbm.at[pl.ds(cbase, CH)], w_b)

        def batch_body(b, c1):
            if scale:
                pltpu.async_copy(x_hbm.at[src_b.at[b]], rows, sem).wait()

            if scale:
                def scale_group(g, c2):
                    wv = w_b[b, pl.ds(g * LANES, LANES)]
                    e0 = g * LANES
                    for l in range(LANES):
                        ws = wv[l]
                        for j in range(D // LANES):
                            sl = pl.ds(j * LANES, LANES)
                            rows[e0 + l, sl] = rows[e0 + l, sl] * ws
                    return c2
                lax.fori_loop(0, EB // LANES, scale_group, 0)

            pltpu.sync_copy(rows, agg_sh.at[dst_b.at[b]], add=True)
            return c1
        lax.fori_loop(0, CH, batch_body, 0)
        return carry
    lax.fori_loop(0, RPW // CH, chunk_body, 0)

    plsc.subcore_barrier()

    for c in range(RPS // EB):
        coff = off0 + c * EB
        build_idx(coff)
        pltpu.sync_copy(agg_sh.at[src_b.at[0]], rows)
        pltpu.sync_copy(rows, agg_out.at[cid, pl.ds(coff, EB)])


def _make_sc(scale):
    import functools
    mesh = plsc.VectorSubcoreMesh(core_axis_name="c", subcore_axis_name="s")
    return pl.kernel(
        functools.partial(_sc_body, scale),
        out_type=jax.ShapeDtypeStruct((NC, NPAD, D), jnp.float32),
        mesh=mesh,
        scratch_types=[
            pltpu.VMEM((CH, EB), jnp.int32),
            pltpu.VMEM((CH, EB), jnp.int32),
            pltpu.VMEM((CH, EB), jnp.float32),
            pltpu.VMEM((EB, D), jnp.float32),
            pltpu.VMEM_SHARED((NPAD, D), jnp.float32),
            pltpu.SemaphoreType.DMA,
        ],
    )


def _tc_body(agg_ref, cnt_ref, x_ref, wr_ref, wroot_ref, br_ref, out_ref):
    agg = agg_ref[0] + agg_ref[1]
    cnt = cnt_ref[0, :, :1] + cnt_ref[1, :, :1]
    mean = agg / jnp.maximum(cnt, 1.0)
    out_ref[...] = (
        lax.dot_general(mean, wr_ref[...], (((1,), (1,)), ((), ())),
                        preferred_element_type=jnp.float32)
        + lax.dot_general(x_ref[...], wroot_ref[...], (((1,), (1,)), ((), ())),
                          preferred_element_type=jnp.float32)
        + br_ref[...]
    )


def _tc_combine(agg2, cnt2, x2d, W_rel, b_rel2, W_root):
    R = 1000
    return pl.pallas_call(
        _tc_body,
        grid=(N // R,),
        in_specs=[
            pl.BlockSpec((NC, R, D), lambda i: (0, i, 0)),
            pl.BlockSpec((NC, R, D), lambda i: (0, i, 0)),
            pl.BlockSpec((R, D), lambda i: (i, 0)),
            pl.BlockSpec((D, D), lambda i: (0, 0)),
            pl.BlockSpec((D, D), lambda i: (0, 0)),
            pl.BlockSpec((1, D), lambda i: (0, 0)),
        ],
        out_specs=pl.BlockSpec((R, D), lambda i: (i, 0)),
        out_shape=jax.ShapeDtypeStruct((N, D), jnp.float32),
    )(agg2, cnt2, x2d, W_rel, W_root, b_rel2)


@jax.jit
def kernel(x, index, weight, W_rel, b_rel, W_root):
    x2d = x.reshape(N, D)
    pad = NW * RPW * EB - E
    src2d = jnp.pad(index[0], (0, pad)).reshape(NW * RPW, EB)
    dst2d = jnp.pad(index[1], (0, pad), constant_values=N).reshape(NW * RPW, EB)
    w2d = jnp.pad(weight, (0, pad)).reshape(NW * RPW, EB)

    agg2 = _make_sc(True)(x2d, src2d, dst2d, w2d)
    cnt2 = _make_sc(False)(x2d, src2d, dst2d, w2d)
    out = _tc_combine(agg2, cnt2, x2d, W_rel, b_rel.reshape(1, D), W_root)
    return out.reshape(1, N, D)
